# Initial kernel scaffold; baseline (speedup 1.0000x reference)
import jax
import jax.numpy as jnp
from jax.experimental import pallas as pl


def kernel(x, xyz, Wq, bq, Wk, bk, Wv, bv, Wo, bo, Wb1, bb1, Wb2, bb2):
    raise NotImplementedError("write your pallas kernel here")


# speedup vs baseline: 4.6574x; 4.6574x over previous
import functools
import math

import jax
import jax.numpy as jnp
from jax import lax
from jax.experimental import pallas as pl
from jax.experimental.pallas import tpu as pltpu
from jax.experimental.pallas import tpu_sc as plsc

DIM = 512
H = 16
K = 32
HD = DIM // H

_TQ = 1024
_TN = 256
_TA = 128
_SC_C = 64


def _qkv_body(x_ref, wq_ref, wk_ref, wv_ref, bq_ref, bk_ref, bv_ref,
              q_ref, k_ref, v_ref):
    xt = x_ref[...]
    q_ref[...] = jnp.dot(xt, wq_ref[...],
                         preferred_element_type=jnp.float32) + bq_ref[...]
    k_ref[...] = jnp.dot(xt, wk_ref[...],
                         preferred_element_type=jnp.float32) + bk_ref[...]
    v_ref[...] = jnp.dot(xt, wv_ref[...],
                         preferred_element_type=jnp.float32) + bv_ref[...]


def _qkv(xf, wqT, wkT, wvT, bq2, bk2, bv2):
    bn = xf.shape[0]
    grid = (bn // _TQ,)
    row = pl.BlockSpec((_TQ, DIM), lambda i: (i, 0))
    w = pl.BlockSpec((DIM, DIM), lambda i: (0, 0))
    b = pl.BlockSpec((1, DIM), lambda i: (0, 0))
    out = jax.ShapeDtypeStruct((bn, DIM), jnp.float32)
    return pl.pallas_call(
        _qkv_body,
        grid=grid,
        in_specs=[row, w, w, w, b, b, b],
        out_specs=[row, row, row],
        out_shape=[out, out, out],
    )(xf, wqT, wkT, wvT, bq2, bk2, bv2)


def _knn_body(n, xyz_ref, xyzT_ref, idx_ref):
    bidx = pl.program_id(0)
    xt = xyz_ref[0]
    xa = xyzT_ref[0]
    x2t = jnp.sum(xt * xt, axis=1, keepdims=True)
    x2a = jnp.sum(xa * xa, axis=0, keepdims=True)
    cross = jax.lax.dot_general(
        xt.astype(jnp.bfloat16), xa.astype(jnp.bfloat16),
        (((1,), (0,)), ((), ())),
        preferred_element_type=jnp.float32)
    d2 = jnp.maximum(x2t + x2a - 2.0 * cross, 0.0)
    iota = lax.broadcasted_iota(jnp.int32, (_TN, n), 1)
    inf = jnp.float32(jnp.inf)
    cols = []
    for _ in range(K):
        m = jnp.min(d2, axis=1, keepdims=True)
        miota = jnp.where(d2 <= m, iota, n)
        am = jnp.min(miota, axis=1, keepdims=True)
        cols.append(am)
        d2 = jnp.where(iota == am, inf, d2)
    idx_ref[0] = jnp.concatenate(cols, axis=1) + bidx * n


def _knn(xyzp8, xyzT):
    bsz, n = xyzp8.shape[0], xyzp8.shape[1]
    grid = (bsz, n // _TN)
    return pl.pallas_call(
        functools.partial(_knn_body, n),
        grid=grid,
        in_specs=[
            pl.BlockSpec((1, _TN, 8), lambda bi, i: (bi, i, 0)),
            pl.BlockSpec((1, 8, n), lambda bi, i: (bi, 0, 0)),
        ],
        out_specs=pl.BlockSpec((1, _TN, K), lambda bi, i: (bi, i, 0)),
        out_shape=jax.ShapeDtypeStruct((bsz, n, K), jnp.int32),
    )(xyzp8, xyzT)


def _make_gather(n_idx):
    mesh = plsc.VectorSubcoreMesh(core_axis_name="c", subcore_axis_name="s")
    nw = 32
    per_w = n_idx // nw
    n_it = per_w // _SC_C

    @functools.partial(
        pl.kernel,
        out_type=(
            jax.ShapeDtypeStruct((n_idx, DIM), jnp.float32),
            jax.ShapeDtypeStruct((n_idx, DIM), jnp.float32),
            jax.ShapeDtypeStruct((n_idx, 128), jnp.float32),
        ),
        mesh=mesh,
        scratch_types=[
            pltpu.VMEM((_SC_C,), jnp.int32),
            pltpu.VMEM((_SC_C, DIM), jnp.float32),
            pltpu.VMEM((_SC_C, DIM), jnp.float32),
            pltpu.VMEM((_SC_C, 128), jnp.float32),
            pltpu.SemaphoreType.DMA,
            pltpu.SemaphoreType.DMA,
            pltpu.SemaphoreType.DMA,
        ],
    )
    def gather_k(ktab, vtab, xtab, idx_hbm, knb, vnb, xnb,
                 idx_v, rk, rv, rx, s1, s2, s3):
        wid = lax.axis_index("s") * 2 + lax.axis_index("c")

        def body(i, carry):
            base = wid * per_w + i * _SC_C
            pltpu.sync_copy(idx_hbm.at[pl.ds(base, _SC_C)], idx_v)
            ck = pltpu.async_copy(ktab.at[idx_v], rk, s1)
            cv = pltpu.async_copy(vtab.at[idx_v], rv, s2)
            cx = pltpu.async_copy(xtab.at[idx_v], rx, s3)
            ck.wait()
            cv.wait()
            cx.wait()
            pltpu.sync_copy(rk, knb.at[pl.ds(base, _SC_C)])
            pltpu.sync_copy(rv, vnb.at[pl.ds(base, _SC_C)])
            pltpu.sync_copy(rx, xnb.at[pl.ds(base, _SC_C)])
            return carry

        lax.fori_loop(0, n_it, body, 0)

    return gather_k


def _attn_body(q_ref, xyz_ref, knb_ref, vnb_ref, nbx_ref,
               wb1_ref, bb1_ref, wb2_ref, bb2_ref, wo_ref, bo_ref, y_ref):
    q = q_ref[...]
    xyz = xyz_ref[...]
    nbx = nbx_ref[...]
    rel = xyz[:, None, :] - nbx
    h1 = jnp.broadcast_to(bb1_ref[0:1, :][None], (_TA, K, 32))
    for a in range(3):
        h1 = h1 + rel[:, :, a:a + 1] * wb1_ref[a:a + 1, :][None, :, :]
    h1 = jnp.maximum(h1, 0.0)

    knb = knb_ref[...]
    vnb = vnb_ref[...]
    scale = jnp.float32(1.0 / math.sqrt(HD))
    outs = []
    for h in range(H):
        qh = q[:, h * HD:(h + 1) * HD]
        kh = knb[:, :, h * HD:(h + 1) * HD]
        s = jnp.sum(kh * qh[:, None, :], axis=2, keepdims=True) * scale
        bh = jnp.sum(h1 * wb2_ref[h:h + 1, :][None, :, :], axis=2,
                     keepdims=True) + bb2_ref[0:1, h:h + 1][None]
        s = s + bh
        m = jnp.max(s, axis=1, keepdims=True)
        p = jnp.exp(s - m)
        l = jnp.sum(p, axis=1, keepdims=True)
        att = p / l
        vh = vnb[:, :, h * HD:(h + 1) * HD]
        outs.append(jnp.sum(vh * att, axis=1))
    o = jnp.concatenate(outs, axis=1)
    y_ref[...] = jnp.dot(o, wo_ref[...],
                         preferred_element_type=jnp.float32) + bo_ref[...]


def _attn(q, xyzp16, knb, vnb, xnb, wb1p, bb1_2, Wb2, bb2_2, woT, bo2):
    bn = q.shape[0]
    grid = (bn // _TA,)
    row = pl.BlockSpec((_TA, DIM), lambda i: (i, 0))
    return pl.pallas_call(
        _attn_body,
        grid=grid,
        in_specs=[
            row,
            pl.BlockSpec((_TA, 128), lambda i: (i, 0)),
            pl.BlockSpec((_TA, K, DIM), lambda i: (i, 0, 0)),
            pl.BlockSpec((_TA, K, DIM), lambda i: (i, 0, 0)),
            pl.BlockSpec((_TA, K, 128), lambda i: (i, 0, 0)),
            pl.BlockSpec((16, 32), lambda i: (0, 0)),
            pl.BlockSpec((1, 32), lambda i: (0, 0)),
            pl.BlockSpec((H, 32), lambda i: (0, 0)),
            pl.BlockSpec((1, H), lambda i: (0, 0)),
            pl.BlockSpec((DIM, DIM), lambda i: (0, 0)),
            pl.BlockSpec((1, DIM), lambda i: (0, 0)),
        ],
        out_specs=row,
        out_shape=jax.ShapeDtypeStruct((bn, DIM), jnp.float32),
    )(q, xyzp16, knb, vnb, xnb, wb1p, bb1_2, Wb2, bb2_2, woT, bo2)


def kernel(x, xyz, Wq, bq, Wk, bk, Wv, bv, Wo, bo, Wb1, bb1, Wb2, bb2):
    bsz, n, c = x.shape
    bn = bsz * n
    xf = x.reshape(bn, c)

    q, kf, vf = _qkv(xf, Wq.T, Wk.T, Wv.T, bq[None], bk[None], bv[None])

    xyzp8 = jnp.pad(xyz, ((0, 0), (0, 0), (0, 5)))
    xyzT = jnp.swapaxes(xyzp8, 1, 2)
    idx = _knn(xyzp8, xyzT)
    idx_flat = idx.reshape(bn * K)

    xyzp128 = jnp.pad(xyz.reshape(bn, 3), ((0, 0), (0, 125)))
    knb, vnb, xnb = _make_gather(bn * K)(kf, vf, xyzp128, idx_flat)

    wb1p = jnp.pad(Wb1.T, ((0, 13), (0, 0)))
    y = _attn(q, xyzp128,
              knb.reshape(bn, K, DIM), vnb.reshape(bn, K, DIM),
              xnb.reshape(bn, K, 128),
              wb1p, bb1[None], Wb2, bb2[None], Wo.T, bo[None])
    return y.reshape(bsz, n, c)


# speedup vs baseline: 6.1160x; 1.3132x over previous
import functools
import math

import jax
import jax.numpy as jnp
from jax import lax
from jax.experimental import pallas as pl
from jax.experimental.pallas import tpu as pltpu
from jax.experimental.pallas import tpu_sc as plsc

DIM = 512
H = 16
K = 32
HD = DIM // H

_TQ = 1024
_TN = 256
_TA = 128
_SC_C = 64


def _qkv_body(x_ref, wq_ref, wk_ref, wv_ref, bq_ref, bk_ref, bv_ref,
              q_ref, k_ref, v_ref):
    xt = x_ref[...]
    q_ref[...] = jnp.dot(xt, wq_ref[...],
                         preferred_element_type=jnp.float32) + bq_ref[...]
    k_ref[...] = jnp.dot(xt, wk_ref[...],
                         preferred_element_type=jnp.float32) + bk_ref[...]
    v_ref[...] = jnp.dot(xt, wv_ref[...],
                         preferred_element_type=jnp.float32) + bv_ref[...]


def _qkv(xf, wqT, wkT, wvT, bq2, bk2, bv2):
    bn = xf.shape[0]
    grid = (bn // _TQ,)
    row = pl.BlockSpec((_TQ, DIM), lambda i: (i, 0))
    w = pl.BlockSpec((DIM, DIM), lambda i: (0, 0))
    b = pl.BlockSpec((1, DIM), lambda i: (0, 0))
    out = jax.ShapeDtypeStruct((bn, DIM), jnp.float32)
    return pl.pallas_call(
        _qkv_body,
        grid=grid,
        in_specs=[row, w, w, w, b, b, b],
        out_specs=[row, row, row],
        out_shape=[out, out, out],
    )(xf, wqT, wkT, wvT, bq2, bk2, bv2)


def _knn_body(n, xyz_ref, xyzT_ref, idx_ref):
    bidx = pl.program_id(0)
    xt = xyz_ref[0]
    xa = xyzT_ref[0]
    x2t = jnp.sum(xt * xt, axis=1, keepdims=True)
    x2a = jnp.sum(xa * xa, axis=0, keepdims=True)
    cross = jax.lax.dot_general(
        xt.astype(jnp.bfloat16), xa.astype(jnp.bfloat16),
        (((1,), (0,)), ((), ())),
        preferred_element_type=jnp.float32)
    d2 = jnp.maximum(x2t + x2a - 2.0 * cross, 0.0)
    iota = lax.broadcasted_iota(jnp.int32, (_TN, n), 1)
    inf = jnp.float32(jnp.inf)
    cols = []
    for _ in range(K):
        m = jnp.min(d2, axis=1, keepdims=True)
        miota = jnp.where(d2 <= m, iota, n)
        am = jnp.min(miota, axis=1, keepdims=True)
        cols.append(am)
        d2 = jnp.where(iota == am, inf, d2)
    idx_ref[0] = jnp.concatenate(cols, axis=1) + bidx * n


def _knn(xyzp8, xyzT):
    bsz, n = xyzp8.shape[0], xyzp8.shape[1]
    grid = (bsz, n // _TN)
    return pl.pallas_call(
        functools.partial(_knn_body, n),
        grid=grid,
        in_specs=[
            pl.BlockSpec((1, _TN, 8), lambda bi, i: (bi, i, 0)),
            pl.BlockSpec((1, 8, n), lambda bi, i: (bi, 0, 0)),
        ],
        out_specs=pl.BlockSpec((1, _TN, K), lambda bi, i: (bi, i, 0)),
        out_shape=jax.ShapeDtypeStruct((bsz, n, K), jnp.int32),
    )(xyzp8, xyzT)


def _make_gather(n_idx):
    mesh = plsc.VectorSubcoreMesh(core_axis_name="c", subcore_axis_name="s")
    nw = 32
    per_w = n_idx // nw
    n_it = per_w // _SC_C

    @functools.partial(
        pl.kernel,
        out_type=(
            jax.ShapeDtypeStruct((n_idx, DIM), jnp.float32),
            jax.ShapeDtypeStruct((n_idx, DIM), jnp.float32),
            jax.ShapeDtypeStruct((n_idx, 128), jnp.float32),
        ),
        mesh=mesh,
        scratch_types=[
            pltpu.VMEM((_SC_C,), jnp.int32),
            pltpu.VMEM((_SC_C, DIM), jnp.float32),
            pltpu.VMEM((_SC_C, DIM), jnp.float32),
            pltpu.VMEM((_SC_C, 128), jnp.float32),
            pltpu.SemaphoreType.DMA,
            pltpu.SemaphoreType.DMA,
            pltpu.SemaphoreType.DMA,
        ],
    )
    def gather_k(ktab, vtab, xtab, idx_hbm, knb, vnb, xnb,
                 idx_v, rk, rv, rx, s1, s2, s3):
        wid = lax.axis_index("s") * 2 + lax.axis_index("c")

        def body(i, carry):
            base = wid * per_w + i * _SC_C
            pltpu.sync_copy(idx_hbm.at[pl.ds(base, _SC_C)], idx_v)
            ck = pltpu.async_copy(ktab.at[idx_v], rk, s1)
            cv = pltpu.async_copy(vtab.at[idx_v], rv, s2)
            cx = pltpu.async_copy(xtab.at[idx_v], rx, s3)
            ck.wait()
            cv.wait()
            cx.wait()
            pltpu.sync_copy(rk, knb.at[pl.ds(base, _SC_C)])
            pltpu.sync_copy(rv, vnb.at[pl.ds(base, _SC_C)])
            pltpu.sync_copy(rx, xnb.at[pl.ds(base, _SC_C)])
            return carry

        lax.fori_loop(0, n_it, body, 0)

    return gather_k


def _dot(a, b, prec=None):
    return jax.lax.dot_general(a, b, (((1,), (0,)), ((), ())),
                               precision=prec,
                               preferred_element_type=jnp.float32)


def _attn_body(q_ref, xyz_ref, knb_ref, vnb_ref, nbx_ref,
               wb1_ref, bb1_ref, wb2_ref, bb2_ref, wo_ref, bo_ref,
               bd_ref, bdt_ref, y_ref):
    hi = jax.lax.Precision.HIGHEST
    scale = jnp.float32(1.0 / math.sqrt(HD))
    q = q_ref[...] * scale
    xyz = xyz_ref[...]
    nbx = nbx_ref[...]
    rel = (xyz[:, None, :] - nbx).reshape(_TA * K, 128)
    h1 = jnp.maximum(_dot(rel, wb1_ref[...], hi) + bb1_ref[...], 0.0)
    bias2 = _dot(h1, wb2_ref[...], hi) + bb2_ref[...]

    knb = knb_ref[...]
    vnb = vnb_ref[...]
    prod = (knb * q[:, None, :]).reshape(_TA * K, DIM)
    s2 = _dot(prod, bd_ref[...], hi) + bias2
    s3 = s2.reshape(_TA, K, H)
    m = jnp.max(s3, axis=1, keepdims=True)
    p = jnp.exp(s3 - m)
    l = jnp.sum(p, axis=1, keepdims=True)
    a2 = (p / l).reshape(_TA * K, H)
    aexp = _dot(a2, bdt_ref[...], hi).reshape(_TA, K, DIM)
    o = jnp.sum(aexp * vnb, axis=1)
    y_ref[...] = jnp.dot(o, wo_ref[...],
                         preferred_element_type=jnp.float32) + bo_ref[...]


def _attn(q, xyzp16, knb, vnb, xnb, wb1p, bb1_2, wb2T, bb2_2, woT, bo2, bd, bdt):
    bn = q.shape[0]
    grid = (bn // _TA,)
    row = pl.BlockSpec((_TA, DIM), lambda i: (i, 0))
    return pl.pallas_call(
        _attn_body,
        grid=grid,
        in_specs=[
            row,
            pl.BlockSpec((_TA, 128), lambda i: (i, 0)),
            pl.BlockSpec((_TA, K, DIM), lambda i: (i, 0, 0)),
            pl.BlockSpec((_TA, K, DIM), lambda i: (i, 0, 0)),
            pl.BlockSpec((_TA, K, 128), lambda i: (i, 0, 0)),
            pl.BlockSpec((128, 32), lambda i: (0, 0)),
            pl.BlockSpec((1, 32), lambda i: (0, 0)),
            pl.BlockSpec((32, H), lambda i: (0, 0)),
            pl.BlockSpec((1, H), lambda i: (0, 0)),
            pl.BlockSpec((DIM, DIM), lambda i: (0, 0)),
            pl.BlockSpec((1, DIM), lambda i: (0, 0)),
            pl.BlockSpec((DIM, H), lambda i: (0, 0)),
            pl.BlockSpec((H, DIM), lambda i: (0, 0)),
        ],
        out_specs=row,
        out_shape=jax.ShapeDtypeStruct((bn, DIM), jnp.float32),
    )(q, xyzp16, knb, vnb, xnb, wb1p, bb1_2, wb2T, bb2_2, woT, bo2, bd, bdt)


def kernel(x, xyz, Wq, bq, Wk, bk, Wv, bv, Wo, bo, Wb1, bb1, Wb2, bb2):
    bsz, n, c = x.shape
    bn = bsz * n
    xf = x.reshape(bn, c)

    q, kf, vf = _qkv(xf, Wq.T, Wk.T, Wv.T, bq[None], bk[None], bv[None])

    xyzp8 = jnp.pad(xyz, ((0, 0), (0, 0), (0, 5)))
    xyzT = jnp.swapaxes(xyzp8, 1, 2)
    idx = _knn(xyzp8, xyzT)
    idx_flat = idx.reshape(bn * K)

    xyzp128 = jnp.pad(xyz.reshape(bn, 3), ((0, 0), (0, 125)))
    knb, vnb, xnb = _make_gather(bn * K)(kf, vf, xyzp128, idx_flat)

    wb1p = jnp.pad(Wb1.T, ((0, 125), (0, 0)))
    eye = jnp.eye(H, dtype=jnp.float32)
    bd = jnp.repeat(eye, HD, axis=0)
    y = _attn(q, xyzp128,
              knb.reshape(bn, K, DIM), vnb.reshape(bn, K, DIM),
              xnb.reshape(bn, K, 128),
              wb1p, bb1[None], Wb2.T, bb2[None], Wo.T, bo[None],
              bd, bd.T)
    return y.reshape(bsz, n, c)
